# baseline (speedup 1.0000x reference)
import functools

import jax
import jax.numpy as jnp
from jax import lax
from jax.experimental import pallas as pl
from jax.experimental.pallas import tpu as pltpu
from jax.experimental.pallas import tpu_sc as plsc

N = 10000
E = 320000
D = 128
N_PAD = 10240
NC = 2
NS = 16
NW = NC * NS
EW = E // NW
PASS = 25
NBUF = 4
ROWS_PER_TILE = N // NS
ZR = 5
K1C = 80
K1NPASS = 2 * E // (NW * K1C * PASS)
K5C = 80
K5NPASS = EW // (K5C * PASS)
K5RING = 4


@functools.cache
def _mesh_kwargs():
    return dict(
        mesh=plsc.VectorSubcoreMesh(
            core_axis_name="c", subcore_axis_name="s", num_cores=NC, num_subcores=NS
        )
    )


def _zero_fill(ref, nrows, width):
    @pl.loop(0, nrows)
    def _(i):
        for k in range(width // 16):
            ref[i, pl.ds(k * 16, 16)] = jnp.zeros((16,), jnp.float32)


def _k1_body(src_hbm, dst_hbm, out_hbm, sidx, didx, ones_v, zz_v, deg_sh, *sems):
    c = lax.axis_index("c")
    s = lax.axis_index("s")
    wid = c * NS + s
    for k in range(K1C // 16):
        ones_v[pl.ds(k * 16, 16)] = jnp.full((16,), 1.0, jnp.float32)
    nz = 2 * N_PAD // NS
    @pl.loop(0, 10)
    def _(i):
        zz_v[pl.ds(i * 16, 16)] = jnp.zeros((16,), jnp.float32)
    @pl.loop(0, nz // 160)
    def _(i):
        pltpu.sync_copy(zz_v, deg_sh.at[pl.ds(s * nz + i * 160, 160)])
    plsc.subcore_barrier()
    dego = deg_sh.at[pl.ds(0, N_PAD)]
    degi = deg_sh.at[pl.ds(N_PAD, N_PAD)]

    def fire(table, idx, j, b):
        pltpu.async_copy(ones_v, table.at[idx.at[j]], sems[b], add=True)

    def drain(b):
        pltpu.make_async_copy(
            out_hbm.at[0, pl.ds(0, K1C)], ones_v, sems[b]
        ).wait()

    @pl.loop(0, K5NPASS)
    def _(p):
        pltpu.sync_copy(src_hbm.at[wid, p], sidx)
        pltpu.sync_copy(dst_hbm.at[wid, p], didx)
        for j in range(5):
            fire(dego, sidx, j, j)
            fire(degi, didx, j, j + 5)
        @pl.loop(0, PASS // 5 - 1)
        def _(g):
            for i in range(5):
                j = 5 + g * 5 + i
                drain(i)
                fire(dego, sidx, j, i)
                drain(i + 5)
                fire(degi, didx, j, i + 5)
        for b in range(10):
            drain(b)
    plsc.subcore_barrier()
    pltpu.sync_copy(deg_sh.at[pl.ds(s * nz, nz)], out_hbm.at[c, pl.ds(s * nz, nz)])


@functools.cache
def _get_k1():
    return pl.kernel(
        _k1_body,
        out_type=jax.ShapeDtypeStruct((NC, 2 * N_PAD), jnp.float32),
        scratch_types=[
            pltpu.VMEM((PASS, K1C), jnp.int32),
            pltpu.VMEM((PASS, K1C), jnp.int32),
            pltpu.VMEM((K1C,), jnp.float32),
            pltpu.VMEM((160,), jnp.float32),
            pltpu.VMEM_SHARED((2 * N_PAD,), jnp.float32),
        ]
        + [pltpu.SemaphoreType.DMA] * 10,
        compiler_params=pltpu.CompilerParams(use_tc_tiling_on_sc=False, skip_device_barrier=True),
        **_mesh_kwargs(),
    )


@functools.cache
def _make_seg_kernel(width, chunk, nbuf):
    npass = EW // (chunk * PASS)

    def body(table_hbm, src_hbm, dst_hbm, out_hbm, sidx, didx, zrow, acc_sh, *rest):
        bufs = rest[:nbuf]
        gsems = rest[nbuf:2 * nbuf]
        ssems = rest[2 * nbuf:3 * nbuf]
        c = lax.axis_index("c")
        s = lax.axis_index("s")
        wid = c * NS + s
        _zero_fill(zrow, ZR, width)
        base = s * ROWS_PER_TILE
        @pl.loop(0, ROWS_PER_TILE // ZR)
        def _(i):
            pltpu.sync_copy(zrow, acc_sh.at[pl.ds(base + i * ZR, ZR)])
        plsc.subcore_barrier()

        def wait_g(b):
            pltpu.make_async_copy(table_hbm.at[sidx.at[0]], bufs[b], gsems[b]).wait()

        def wait_s(b):
            pltpu.make_async_copy(table_hbm.at[sidx.at[0]], bufs[b], ssems[b]).wait()

        nsteady = (PASS - nbuf) // nbuf * nbuf
        @pl.loop(0, npass)
        def _(p):
            pltpu.sync_copy(src_hbm.at[wid, p], sidx)
            pltpu.sync_copy(dst_hbm.at[wid, p], didx)
            for b in range(nbuf):
                pltpu.async_copy(table_hbm.at[sidx.at[b]], bufs[b], gsems[b])
            wait_g(0)
            pltpu.async_copy(bufs[0], acc_sh.at[didx.at[0]], ssems[0], add=True)

            def step(j, b, bprev, refire_ok):
                wait_g(b)
                pltpu.async_copy(bufs[b], acc_sh.at[didx.at[j]], ssems[b], add=True)
                wait_s(bprev)
                if refire_ok:
                    pltpu.async_copy(
                        table_hbm.at[sidx.at[j - 1 + nbuf]], bufs[bprev], gsems[bprev]
                    )

            @pl.loop(0, nsteady // nbuf)
            def _(g):
                for i in range(nbuf):
                    step(1 + g * nbuf + i, (1 + i) % nbuf, i % nbuf, True)
            for j in range(1 + nsteady, PASS):
                step(j, j % nbuf, (j - 1) % nbuf, j - 1 + nbuf < PASS)
            wait_s((PASS - 1) % nbuf)
        plsc.subcore_barrier()
        pltpu.sync_copy(
            acc_sh.at[pl.ds(base, ROWS_PER_TILE)],
            out_hbm.at[c, pl.ds(base, ROWS_PER_TILE)],
        )

    return pl.kernel(
        body,
        out_type=jax.ShapeDtypeStruct((NC, N, width), jnp.float32),
        scratch_types=[
            pltpu.VMEM((PASS, chunk), jnp.int32),
            pltpu.VMEM((PASS, chunk), jnp.int32),
            pltpu.VMEM((ZR, width), jnp.float32),
            pltpu.VMEM_SHARED((N, width), jnp.float32),
        ]
        + [pltpu.VMEM((chunk, width), jnp.float32)] * nbuf
        + [pltpu.SemaphoreType.DMA] * (2 * nbuf),
        compiler_params=pltpu.CompilerParams(use_tc_tiling_on_sc=False, skip_device_barrier=True),
        **_mesh_kwargs(),
    )


def _k5_body(y_hbm, src_hbm, dst_hbm, out_hbm, ytab, sidx, didx, vals, zz_v, acc_sh,
             *ssems):
    c = lax.axis_index("c")
    s = lax.axis_index("s")
    wid = c * NS + s
    @pl.loop(0, 10)
    def _(i):
        zz_v[pl.ds(i * 16, 16)] = jnp.zeros((16,), jnp.float32)
    pltpu.sync_copy(y_hbm, ytab)
    @pl.loop(0, 4)
    def _(i):
        pltpu.sync_copy(zz_v, acc_sh.at[pl.ds(s * 640 + i * 160, 160)])
    plsc.subcore_barrier()

    def wait_s(b):
        pltpu.make_async_copy(y_hbm.at[pl.ds(0, K5C)], vals.at[b], ssems[b]).wait()

    def fill_fire(j, b):
        for k in range(K5C // 16):
            idxv = sidx[j, pl.ds(k * 16, 16)]
            vals[b, pl.ds(k * 16, 16)] = plsc.load_gather(ytab, [idxv])
        pltpu.async_copy(vals.at[b], acc_sh.at[didx.at[j]], ssems[b], add=True)

    @pl.loop(0, K5NPASS)
    def _(p):
        pltpu.sync_copy(src_hbm.at[wid, p], sidx)
        pltpu.sync_copy(dst_hbm.at[wid, p], didx)
        for j in range(K5RING):
            fill_fire(j, j)
        @pl.loop(0, (PASS - K5RING) // K5RING)
        def _(g):
            for i in range(K5RING):
                j = K5RING + g * K5RING + i
                wait_s(i)
                fill_fire(j, i)
        for j in range(K5RING + (PASS - K5RING) // K5RING * K5RING, PASS):
            b = j % K5RING
            wait_s(b)
            fill_fire(j, b)
        for jj in range(PASS - K5RING, PASS):
            wait_s(jj % K5RING)
    plsc.subcore_barrier()
    pltpu.sync_copy(
        acc_sh.at[pl.ds(s * 640, 640)], out_hbm.at[c, pl.ds(s * 640, 640)]
    )


@functools.cache
def _get_k5():
    return pl.kernel(
        _k5_body,
        out_type=jax.ShapeDtypeStruct((NC, N_PAD), jnp.float32),
        scratch_types=[
            pltpu.VMEM((N,), jnp.float32),
            pltpu.VMEM((PASS, K5C), jnp.int32),
            pltpu.VMEM((PASS, K5C), jnp.int32),
            pltpu.VMEM((K5RING, K5C), jnp.float32),
            pltpu.VMEM((160,), jnp.float32),
            pltpu.VMEM_SHARED((N_PAD,), jnp.float32),
        ]
        + [pltpu.SemaphoreType.DMA] * K5RING,
        compiler_params=pltpu.CompilerParams(
            use_tc_tiling_on_sc=False, needs_layout_passes=False,
            skip_device_barrier=True,
        ),
        **_mesh_kwargs(),
    )


_R = 1000
_G = N // _R


def _k2_body(deg_ref, x_ref, xs_ref, nrm_ref):
    dsrc = deg_ref[0, 0] + deg_ref[1, 0]
    ddst = deg_ref[0, 1] + deg_ref[1, 1]
    ns = lax.rsqrt(jnp.maximum(dsrc, 1.0))
    nd = lax.rsqrt(jnp.maximum(ddst, 1.0))
    nrm_ref[0] = ns
    nrm_ref[1] = nd
    xs_ref[...] = x_ref[...] * ns


def _k2_norms_prescale(deg4, x):
    return pl.pallas_call(
        _k2_body,
        grid=(_G,),
        in_specs=[
            pl.BlockSpec((NC, 2, _R, 1), lambda i: (0, 0, i, 0)),
            pl.BlockSpec((_R, D), lambda i: (i, 0)),
        ],
        out_specs=[
            pl.BlockSpec((_R, D), lambda i: (i, 0)),
            pl.BlockSpec((2, _R, 1), lambda i: (0, i, 0)),
        ],
        out_shape=[
            jax.ShapeDtypeStruct((N, D), jnp.float32),
            jax.ShapeDtypeStruct((2, N, 1), jnp.float32),
        ],
    )(deg4, x)


def _k4_body(aggp_ref, nrm_ref, w0_ref, b0_ref, w1_ref, y16_ref):
    a = (aggp_ref[0] + aggp_ref[1]) * nrm_ref[1]
    h = jnp.dot(a, w0_ref[...], preferred_element_type=jnp.float32) + b0_ref[...]
    h = jnp.maximum(h, 0.0) * nrm_ref[0]
    y16_ref[...] = jnp.dot(h, w1_ref[...], preferred_element_type=jnp.float32)


def _k4_dense(aggp, nrm, W0, b0, W1):
    return pl.pallas_call(
        _k4_body,
        grid=(_G,),
        in_specs=[
            pl.BlockSpec((NC, _R, D), lambda i: (0, i, 0)),
            pl.BlockSpec((2, _R, 1), lambda i: (0, i, 0)),
            pl.BlockSpec((D, D), lambda i: (0, 0)),
            pl.BlockSpec((1, D), lambda i: (0, 0)),
            pl.BlockSpec((D, 1), lambda i: (0, 0)),
        ],
        out_specs=pl.BlockSpec((_R, 1), lambda i: (i, 0)),
        out_shape=jax.ShapeDtypeStruct((N, 1), jnp.float32),
    )(aggp, nrm, W0, b0, W1)


def _k6_body(qp_ref, nrm_ref, b1_ref, out_ref):
    q = qp_ref[0] + qp_ref[1]
    out_ref[...] = jax.nn.sigmoid(q * nrm_ref[1] + b1_ref[0, 0])


def _k6_output(qp, nrm, b1):
    return pl.pallas_call(
        _k6_body,
        grid=(_G,),
        in_specs=[
            pl.BlockSpec((NC, _R, 1), lambda i: (0, i, 0)),
            pl.BlockSpec((2, _R, 1), lambda i: (0, i, 0)),
            pl.BlockSpec((1, 1), lambda i: (0, 0)),
        ],
        out_specs=pl.BlockSpec((_R, 1), lambda i: (i, 0)),
        out_shape=jax.ShapeDtypeStruct((N, 1), jnp.float32),
    )(qp, nrm, b1)


def kernel(inputs, edge_index, W0, b0, W1, b1):
    x = inputs.astype(jnp.float32)
    src = edge_index[0].astype(jnp.int32)
    dst = edge_index[1].astype(jnp.int32)
    src40 = src.reshape(NW, EW // (40 * PASS), PASS, 40)
    dst40 = dst.reshape(NW, EW // (40 * PASS), PASS, 40)
    src80 = src.reshape(NW, EW // (80 * PASS), PASS, 80)
    dst80 = dst.reshape(NW, EW // (80 * PASS), PASS, 80)
    degp = _get_k1()(src80, dst80)
    deg4 = degp.reshape(NC, 2, N_PAD, 1)
    xs, nrm = _k2_norms_prescale(deg4, x)
    aggp = _make_seg_kernel(D, 40, 5)(xs, src40, dst40)
    y = _k4_dense(aggp, nrm, W0, b0.reshape(1, D), W1)
    qp = _get_k5()(y.reshape(N), src80, dst80)
    out = _k6_output(qp.reshape(NC, N_PAD, 1), nrm, b1.reshape(1, 1))
    return out


# speedup vs baseline: 1.0019x; 1.0019x over previous
import functools

import jax
import jax.numpy as jnp
from jax import lax
from jax.experimental import pallas as pl
from jax.experimental.pallas import tpu as pltpu
from jax.experimental.pallas import tpu_sc as plsc

N = 10000
E = 320000
D = 128
N_PAD = 10240
NC = 2
NS = 16
NW = NC * NS
EW = E // NW
PASS = 25
NBUF = 4
ROWS_PER_TILE = N // NS
ZR = 5
K1C = 80
K1NPASS = 2 * E // (NW * K1C * PASS)
K5C = 80
K5NPASS = EW // (K5C * PASS)
K5RING = 4


@functools.cache
def _mesh_kwargs():
    return dict(
        mesh=plsc.VectorSubcoreMesh(
            core_axis_name="c", subcore_axis_name="s", num_cores=NC, num_subcores=NS
        )
    )


def _zero_fill(ref, nrows, width):
    @pl.loop(0, nrows)
    def _(i):
        for k in range(width // 16):
            ref[i, pl.ds(k * 16, 16)] = jnp.zeros((16,), jnp.float32)


def _k1_body(src_hbm, dst_hbm, out_hbm, sidx, didx, ones_v, zz_v, deg_sh, *sems):
    c = lax.axis_index("c")
    s = lax.axis_index("s")
    wid = c * NS + s
    for k in range(K1C // 16):
        ones_v[pl.ds(k * 16, 16)] = jnp.full((16,), 1.0, jnp.float32)
    nz = 2 * N_PAD // NS
    @pl.loop(0, 10)
    def _(i):
        zz_v[pl.ds(i * 16, 16)] = jnp.zeros((16,), jnp.float32)
    @pl.loop(0, nz // 160)
    def _(i):
        pltpu.sync_copy(zz_v, deg_sh.at[pl.ds(s * nz + i * 160, 160)])
    plsc.subcore_barrier()
    dego = deg_sh.at[pl.ds(0, N_PAD)]
    degi = deg_sh.at[pl.ds(N_PAD, N_PAD)]

    def fire(table, idx, j, b):
        pltpu.async_copy(ones_v, table.at[idx.at[j]], sems[b], add=True)

    def drain(b):
        pltpu.make_async_copy(
            out_hbm.at[0, pl.ds(0, K1C)], ones_v, sems[b]
        ).wait()

    @pl.loop(0, K5NPASS)
    def _(p):
        pltpu.sync_copy(src_hbm.at[wid, p], sidx)
        pltpu.sync_copy(dst_hbm.at[wid, p], didx)
        for j in range(5):
            fire(dego, sidx, j, j)
            fire(degi, didx, j, j + 5)
        @pl.loop(0, PASS // 5 - 1)
        def _(g):
            for i in range(5):
                j = 5 + g * 5 + i
                drain(i)
                fire(dego, sidx, j, i)
                drain(i + 5)
                fire(degi, didx, j, i + 5)
        for b in range(10):
            drain(b)
    plsc.subcore_barrier()
    pltpu.sync_copy(deg_sh.at[pl.ds(s * nz, nz)], out_hbm.at[c, pl.ds(s * nz, nz)])


@functools.cache
def _get_k1():
    return pl.kernel(
        _k1_body,
        out_type=jax.ShapeDtypeStruct((NC, 2 * N_PAD), jnp.float32),
        scratch_types=[
            pltpu.VMEM((PASS, K1C), jnp.int32),
            pltpu.VMEM((PASS, K1C), jnp.int32),
            pltpu.VMEM((K1C,), jnp.float32),
            pltpu.VMEM((160,), jnp.float32),
            pltpu.VMEM_SHARED((2 * N_PAD,), jnp.float32),
        ]
        + [pltpu.SemaphoreType.DMA] * 10,
        compiler_params=pltpu.CompilerParams(use_tc_tiling_on_sc=False),
        **_mesh_kwargs(),
    )


@functools.cache
def _make_seg_kernel(width, chunk, nbuf):
    npass = EW // (chunk * PASS)

    def body(table_hbm, src_hbm, dst_hbm, out_hbm, sidx, didx, zrow, acc_sh, *rest):
        bufs = rest[:nbuf]
        gsems = rest[nbuf:2 * nbuf]
        ssems = rest[2 * nbuf:3 * nbuf]
        c = lax.axis_index("c")
        s = lax.axis_index("s")
        wid = c * NS + s
        _zero_fill(zrow, ZR, width)
        base = s * ROWS_PER_TILE
        @pl.loop(0, ROWS_PER_TILE // ZR)
        def _(i):
            pltpu.sync_copy(zrow, acc_sh.at[pl.ds(base + i * ZR, ZR)])
        plsc.subcore_barrier()

        def wait_g(b):
            pltpu.make_async_copy(table_hbm.at[sidx.at[0]], bufs[b], gsems[b]).wait()

        def wait_s(b):
            pltpu.make_async_copy(table_hbm.at[sidx.at[0]], bufs[b], ssems[b]).wait()

        nsteady = (PASS - nbuf) // nbuf * nbuf
        @pl.loop(0, npass)
        def _(p):
            pltpu.sync_copy(src_hbm.at[wid, p], sidx)
            pltpu.sync_copy(dst_hbm.at[wid, p], didx)
            for b in range(nbuf):
                pltpu.async_copy(table_hbm.at[sidx.at[b]], bufs[b], gsems[b])
            wait_g(0)
            pltpu.async_copy(bufs[0], acc_sh.at[didx.at[0]], ssems[0], add=True)

            def step(j, b, bprev, refire_ok):
                wait_g(b)
                pltpu.async_copy(bufs[b], acc_sh.at[didx.at[j]], ssems[b], add=True)
                wait_s(bprev)
                if refire_ok:
                    pltpu.async_copy(
                        table_hbm.at[sidx.at[j - 1 + nbuf]], bufs[bprev], gsems[bprev]
                    )

            @pl.loop(0, nsteady // nbuf)
            def _(g):
                for i in range(nbuf):
                    step(1 + g * nbuf + i, (1 + i) % nbuf, i % nbuf, True)
            for j in range(1 + nsteady, PASS):
                step(j, j % nbuf, (j - 1) % nbuf, j - 1 + nbuf < PASS)
            wait_s((PASS - 1) % nbuf)
        plsc.subcore_barrier()
        pltpu.sync_copy(
            acc_sh.at[pl.ds(base, ROWS_PER_TILE)],
            out_hbm.at[c, pl.ds(base, ROWS_PER_TILE)],
        )

    return pl.kernel(
        body,
        out_type=jax.ShapeDtypeStruct((NC, N, width), jnp.float32),
        scratch_types=[
            pltpu.VMEM((PASS, chunk), jnp.int32),
            pltpu.VMEM((PASS, chunk), jnp.int32),
            pltpu.VMEM((ZR, width), jnp.float32),
            pltpu.VMEM_SHARED((N, width), jnp.float32),
        ]
        + [pltpu.VMEM((chunk, width), jnp.float32)] * nbuf
        + [pltpu.SemaphoreType.DMA] * (2 * nbuf),
        compiler_params=pltpu.CompilerParams(use_tc_tiling_on_sc=False),
        **_mesh_kwargs(),
    )


def _k5_body(y_hbm, src_hbm, dst_hbm, out_hbm, ytab, sidx, didx, vals, zz_v, acc_sh,
             *ssems):
    c = lax.axis_index("c")
    s = lax.axis_index("s")
    wid = c * NS + s
    @pl.loop(0, 10)
    def _(i):
        zz_v[pl.ds(i * 16, 16)] = jnp.zeros((16,), jnp.float32)
    pltpu.sync_copy(y_hbm, ytab)
    @pl.loop(0, 4)
    def _(i):
        pltpu.sync_copy(zz_v, acc_sh.at[pl.ds(s * 640 + i * 160, 160)])
    plsc.subcore_barrier()

    def wait_s(b):
        pltpu.make_async_copy(y_hbm.at[pl.ds(0, K5C)], vals.at[b], ssems[b]).wait()

    def fill_fire(j, b):
        for k in range(K5C // 16):
            idxv = sidx[j, pl.ds(k * 16, 16)]
            vals[b, pl.ds(k * 16, 16)] = plsc.load_gather(ytab, [idxv])
        pltpu.async_copy(vals.at[b], acc_sh.at[didx.at[j]], ssems[b], add=True)

    @pl.loop(0, K5NPASS)
    def _(p):
        pltpu.sync_copy(src_hbm.at[wid, p], sidx)
        pltpu.sync_copy(dst_hbm.at[wid, p], didx)
        for j in range(K5RING):
            fill_fire(j, j)
        @pl.loop(0, (PASS - K5RING) // K5RING)
        def _(g):
            for i in range(K5RING):
                j = K5RING + g * K5RING + i
                wait_s(i)
                fill_fire(j, i)
        for j in range(K5RING + (PASS - K5RING) // K5RING * K5RING, PASS):
            b = j % K5RING
            wait_s(b)
            fill_fire(j, b)
        for jj in range(PASS - K5RING, PASS):
            wait_s(jj % K5RING)
    plsc.subcore_barrier()
    pltpu.sync_copy(
        acc_sh.at[pl.ds(s * 640, 640)], out_hbm.at[c, pl.ds(s * 640, 640)]
    )


@functools.cache
def _get_k5():
    return pl.kernel(
        _k5_body,
        out_type=jax.ShapeDtypeStruct((NC, N_PAD), jnp.float32),
        scratch_types=[
            pltpu.VMEM((N,), jnp.float32),
            pltpu.VMEM((PASS, K5C), jnp.int32),
            pltpu.VMEM((PASS, K5C), jnp.int32),
            pltpu.VMEM((K5RING, K5C), jnp.float32),
            pltpu.VMEM((160,), jnp.float32),
            pltpu.VMEM_SHARED((N_PAD,), jnp.float32),
        ]
        + [pltpu.SemaphoreType.DMA] * K5RING,
        compiler_params=pltpu.CompilerParams(
            use_tc_tiling_on_sc=False, needs_layout_passes=False
        ),
        **_mesh_kwargs(),
    )


_R = 1000
_G = N // _R


def _k2_body(deg_ref, x_ref, xs_ref, nrm_ref):
    dsrc = deg_ref[0, 0] + deg_ref[1, 0]
    ddst = deg_ref[0, 1] + deg_ref[1, 1]
    ns = lax.rsqrt(jnp.maximum(dsrc, 1.0))
    nd = lax.rsqrt(jnp.maximum(ddst, 1.0))
    nrm_ref[0] = ns
    nrm_ref[1] = nd
    xs_ref[...] = x_ref[...] * ns


def _k2_norms_prescale(deg4, x):
    return pl.pallas_call(
        _k2_body,
        grid=(_G,),
        in_specs=[
            pl.BlockSpec((NC, 2, _R, 1), lambda i: (0, 0, i, 0)),
            pl.BlockSpec((_R, D), lambda i: (i, 0)),
        ],
        out_specs=[
            pl.BlockSpec((_R, D), lambda i: (i, 0)),
            pl.BlockSpec((2, _R, 1), lambda i: (0, i, 0)),
        ],
        out_shape=[
            jax.ShapeDtypeStruct((N, D), jnp.float32),
            jax.ShapeDtypeStruct((2, N, 1), jnp.float32),
        ],
    )(deg4, x)


def _k4_body(aggp_ref, nrm_ref, w0_ref, b0_ref, w1_ref, y16_ref):
    a = (aggp_ref[0] + aggp_ref[1]) * nrm_ref[1]
    h = jnp.dot(a, w0_ref[...], preferred_element_type=jnp.float32) + b0_ref[...]
    h = jnp.maximum(h, 0.0) * nrm_ref[0]
    y16_ref[...] = jnp.dot(h, w1_ref[...], preferred_element_type=jnp.float32)


def _k4_dense(aggp, nrm, W0, b0, W1):
    return pl.pallas_call(
        _k4_body,
        grid=(_G,),
        in_specs=[
            pl.BlockSpec((NC, _R, D), lambda i: (0, i, 0)),
            pl.BlockSpec((2, _R, 1), lambda i: (0, i, 0)),
            pl.BlockSpec((D, D), lambda i: (0, 0)),
            pl.BlockSpec((1, D), lambda i: (0, 0)),
            pl.BlockSpec((D, 1), lambda i: (0, 0)),
        ],
        out_specs=pl.BlockSpec((_R, 1), lambda i: (i, 0)),
        out_shape=jax.ShapeDtypeStruct((N, 1), jnp.float32),
    )(aggp, nrm, W0, b0, W1)


def _k6_body(qp_ref, nrm_ref, b1_ref, out_ref):
    q = qp_ref[0] + qp_ref[1]
    out_ref[...] = jax.nn.sigmoid(q * nrm_ref[1] + b1_ref[0, 0])


def _k6_output(qp, nrm, b1):
    return pl.pallas_call(
        _k6_body,
        grid=(_G,),
        in_specs=[
            pl.BlockSpec((NC, _R, 1), lambda i: (0, i, 0)),
            pl.BlockSpec((2, _R, 1), lambda i: (0, i, 0)),
            pl.BlockSpec((1, 1), lambda i: (0, 0)),
        ],
        out_specs=pl.BlockSpec((_R, 1), lambda i: (i, 0)),
        out_shape=jax.ShapeDtypeStruct((N, 1), jnp.float32),
    )(qp, nrm, b1)


def kernel(inputs, edge_index, W0, b0, W1, b1):
    x = inputs.astype(jnp.float32)
    src = edge_index[0].astype(jnp.int32)
    dst = edge_index[1].astype(jnp.int32)
    src40 = src.reshape(NW, EW // (40 * PASS), PASS, 40)
    dst40 = dst.reshape(NW, EW // (40 * PASS), PASS, 40)
    src80 = src.reshape(NW, EW // (80 * PASS), PASS, 80)
    dst80 = dst.reshape(NW, EW // (80 * PASS), PASS, 80)
    degp = _get_k1()(src80, dst80)
    deg4 = degp.reshape(NC, 2, N_PAD, 1)
    xs, nrm = _k2_norms_prescale(deg4, x)
    aggp = _make_seg_kernel(D, 40, 5)(xs, src40, dst40)
    y = _k4_dense(aggp, nrm, W0, b0.reshape(1, D), W1)
    qp = _get_k5()(y.reshape(N), src80, dst80)
    out = _k6_output(qp.reshape(NC, N_PAD, 1), nrm, b1.reshape(1, 1))
    return out
